# baseline (speedup 1.0000x reference)
import functools

import jax
import jax.numpy as jnp
from jax import lax
from jax.experimental import pallas as pl
from jax.experimental.pallas import tpu as pltpu
from jax.experimental.pallas import tpu_sc as plsc

N = 10000
E = 320000
NP = 10240
G1 = 160
SB = 128
EPT = 40 * 256
EPAD = EPT * 32
RPT = NP // 16

_mesh = plsc.VectorSubcoreMesh(core_axis_name="c", subcore_axis_name="s")
_sc_params = pltpu.CompilerParams(use_tc_tiling_on_sc=False,
                                  needs_layout_passes=False)


def _make_fused(h0, nh, HW, emit_e):
    F = nh * HW * 16
    G = G1
    outs = [jax.ShapeDtypeStruct((2, NP, 16), jnp.float32),
            jax.ShapeDtypeStruct((2, NP, F), jnp.float32)]
    if emit_e:
        outs.append(jax.ShapeDtypeStruct((EPAD, 16), jnp.float32))

    buf = (pltpu.VMEM((G,), jnp.int32),
           pltpu.VMEM((G,), jnp.int32),
           pltpu.VMEM((G, 16), jnp.float32),
           pltpu.VMEM((G, 16), jnp.float32),
           pltpu.VMEM((G, 16), jnp.float32),
           pltpu.VMEM((G, F), jnp.float32),
           pltpu.SemaphoreType.DMA,
           pltpu.SemaphoreType.DMA,
           pltpu.SemaphoreType.DMA)

    @functools.partial(
        pl.kernel,
        out_type=tuple(outs),
        mesh=_mesh,
        scratch_types=buf + buf + (
            pltpu.VMEM((16,), jnp.float32),
            pltpu.VMEM((SB, 16), jnp.float32),
            pltpu.VMEM_SHARED((NP, 16), jnp.float32),
            pltpu.VMEM_SHARED((NP, F), jnp.float32),
        ),
        compiler_params=_sc_params,
    )
    def fused(*refs):
        if emit_e:
            (tas, tad, th, src, dst, shift, zeros16, zerosF,
             dout, oout, eout, *scr) = refs
        else:
            (tas, tad, th, src, dst, shift, zeros16, zerosF,
             dout, oout, *scr) = refs
        bufA, bufB = scr[0:9], scr[9:18]
        shv, stg, den, acc = scr[18:22]
        c = lax.axis_index("c")
        s = lax.axis_index("s")
        wid = c * 16 + s
        r0 = s * RPT
        srows0 = bufA[5]
        pltpu.sync_copy(zeros16, stg)
        pltpu.sync_copy(zerosF, srows0.at[pl.ds(0, SB)])
        for k in range(RPT // SB):
            pltpu.sync_copy(stg, den.at[pl.ds(r0 + k * SB, SB)])
            pltpu.sync_copy(srows0.at[pl.ds(0, SB)],
                            acc.at[pl.ds(r0 + k * SB, SB)])
        pltpu.sync_copy(shift.at[0], shv)
        plsc.subcore_barrier()
        shvv = shv[...]
        ebase = wid * EPT

        def issue(b, buf):
            sidx, didx, asr, adr, eex, srows, s1, s2, s3 = buf
            pltpu.sync_copy(src.at[pl.ds(b, G)], sidx)
            pltpu.sync_copy(dst.at[pl.ds(b, G)], didx)
            c3 = pltpu.async_copy(th.at[sidx], srows, s3)
            c1 = pltpu.async_copy(tas.at[sidx], asr, s1)
            c2 = pltpu.async_copy(tad.at[didx], adr, s2)
            return c1, c2, c3

        def process(b, buf, cps):
            sidx, didx, asr, adr, eex, srows, s1, s2, s3 = buf
            c1, c2, c3 = cps
            c1.wait()
            c2.wait()

            @plsc.parallel_loop(0, G, 1, unroll=4)
            def _edge_e(g):
                pre = asr[g] + adr[g]
                lr = jnp.maximum(pre, 0.2 * pre)
                eex[g] = jnp.exp(lr - shvv)

            c3.wait()

            @plsc.parallel_loop(0, G, 1, unroll=4)
            def _edge_w(g):
                ev = eex[g]
                for h in range(nh):
                    ws = jnp.broadcast_to(ev[h0 + h], (16,))
                    for q in range(HW):
                        sl = pl.ds((h * HW + q) * 16, 16)
                        srows[g, sl] = srows[g, sl] * ws

            if emit_e:
                pltpu.sync_copy(eex, eout.at[pl.ds(b, G)])
            pltpu.sync_copy(eex, den.at[didx], add=True)
            pltpu.sync_copy(srows, acc.at[didx], add=True)

        def pair(i, _):
            b0 = ebase + 2 * i * G
            b1 = b0 + G
            cpa = issue(b0, bufA)
            cpb = issue(b1, bufB)
            process(b0, bufA, cpa)
            process(b1, bufB, cpb)
            return 0

        lax.fori_loop(0, EPT // (2 * G), pair, 0)
        plsc.subcore_barrier()
        for k in range(RPT // SB):
            pltpu.sync_copy(den.at[pl.ds(r0 + k * SB, SB)], stg)
            pltpu.sync_copy(stg, dout.at[c, pl.ds(r0 + k * SB, SB)])
            pltpu.sync_copy(acc.at[pl.ds(r0 + k * SB, SB)],
                            srows0.at[pl.ds(0, SB)])
            pltpu.sync_copy(srows0.at[pl.ds(0, SB)],
                            oout.at[c, pl.ds(r0 + k * SB, SB)])

    return fused


_sc_fused_l1 = _make_fused(0, 6, 1, emit_e=False)
_sc_fused_l2a = _make_fused(0, 3, 2, emit_e=True)


def _make_pass2(h0, nh, HW, G):
    F = nh * HW * 16

    buf = (pltpu.VMEM((G,), jnp.int32),
           pltpu.VMEM((G,), jnp.int32),
           pltpu.VMEM((G, F), jnp.float32),
           pltpu.VMEM((G, 16), jnp.float32),
           pltpu.SemaphoreType.DMA)

    @functools.partial(
        pl.kernel,
        out_type=jax.ShapeDtypeStruct((2, NP, F), jnp.float32),
        mesh=_mesh,
        scratch_types=buf + buf + (
            pltpu.VMEM_SHARED((NP, F), jnp.float32),
        ),
        compiler_params=_sc_params,
    )
    def pass2(th, src, dst, eexp, zerosF, out, *scr):
        bufA, bufB = scr[0:5], scr[5:10]
        acc = scr[10]
        c = lax.axis_index("c")
        s = lax.axis_index("s")
        wid = c * 16 + s
        r0 = s * RPT
        srows0 = bufA[2]
        pltpu.sync_copy(zerosF, srows0.at[pl.ds(0, SB)])
        for k in range(RPT // SB):
            pltpu.sync_copy(srows0.at[pl.ds(0, SB)],
                            acc.at[pl.ds(r0 + k * SB, SB)])
        plsc.subcore_barrier()
        ebase = wid * EPT

        def issue(b, buf):
            sidx, didx, srows, eexc, sem = buf
            pltpu.sync_copy(src.at[pl.ds(b, G)], sidx)
            pltpu.sync_copy(dst.at[pl.ds(b, G)], didx)
            pltpu.sync_copy(eexp.at[pl.ds(b, G)], eexc)
            return pltpu.async_copy(th.at[sidx], srows, sem)

        def process(buf, cp):
            sidx, didx, srows, eexc, sem = buf
            cp.wait()

            @plsc.parallel_loop(0, G, 1, unroll=4)
            def _edge(g):
                ev = eexc[g]
                for h in range(nh):
                    ws = jnp.broadcast_to(ev[h0 + h], (16,))
                    for q in range(HW):
                        sl = pl.ds((h * HW + q) * 16, 16)
                        srows[g, sl] = srows[g, sl] * ws
            pltpu.sync_copy(srows, acc.at[didx], add=True)

        def pair(i, _):
            b0 = ebase + 2 * i * G
            cpa = issue(b0, bufA)
            cpb = issue(b0 + G, bufB)
            process(bufA, cpa)
            process(bufB, cpb)
            return 0

        lax.fori_loop(0, EPT // (2 * G), pair, 0)
        plsc.subcore_barrier()
        for k in range(RPT // SB):
            pltpu.sync_copy(acc.at[pl.ds(r0 + k * SB, SB)],
                            srows0.at[pl.ds(0, SB)])
            pltpu.sync_copy(srows0.at[pl.ds(0, SB)],
                            out.at[c, pl.ds(r0 + k * SB, SB)])

    return pass2


_sc_pass2_l2b = _make_pass2(3, 3, 2, 256)


_GB = 5120
_NG = NP // _GB


def _shift_update(i, tas, tad, sh_ref):
    ms = jnp.max(tas, axis=0)[None, :]
    md = jnp.max(tad, axis=0)[None, :]
    blk = jnp.concatenate([jnp.broadcast_to(ms, (4, 16)),
                           jnp.broadcast_to(md, (4, 16))], axis=0)

    @pl.when(i == 0)
    def _():
        sh_ref[...] = blk

    @pl.when(i > 0)
    def _():
        sh_ref[...] = jnp.maximum(sh_ref[...], blk)

    @pl.when(i == _NG - 1)
    def _():
        a = sh_ref[...]
        tot = a[0:4, :] + a[4:8, :]
        sh = jnp.maximum(tot, 0.2 * tot)
        sh_ref[...] = jnp.concatenate([sh, sh], axis=0)


def _tc_project1(xp, w, was, wad):
    def body(x_ref, w_ref, was_ref, wad_ref,
             th_ref, tas_ref, tad_ref, sh_ref):
        i = pl.program_id(0)
        xb = x_ref[...]
        th = jnp.dot(xb, w_ref[...], preferred_element_type=jnp.float32)
        th_ref[...] = th
        tas = jnp.dot(th, was_ref[...], preferred_element_type=jnp.float32,
                      precision=lax.Precision.HIGHEST)
        tad = jnp.dot(th, wad_ref[...], preferred_element_type=jnp.float32,
                      precision=lax.Precision.HIGHEST)
        tas_ref[...] = tas
        tad_ref[...] = tad
        _shift_update(i, tas, tad, sh_ref)

    return pl.pallas_call(
        body,
        grid=(_NG,),
        in_specs=[pl.BlockSpec((_GB, 16), lambda i: (i, 0)),
                  pl.BlockSpec((16, 96), lambda i: (0, 0)),
                  pl.BlockSpec((96, 16), lambda i: (0, 0)),
                  pl.BlockSpec((96, 16), lambda i: (0, 0))],
        out_specs=[pl.BlockSpec((_GB, 96), lambda i: (i, 0)),
                   pl.BlockSpec((_GB, 16), lambda i: (i, 0)),
                   pl.BlockSpec((_GB, 16), lambda i: (i, 0)),
                   pl.BlockSpec((8, 16), lambda i: (0, 0))],
        out_shape=[jax.ShapeDtypeStruct((NP, 96), jnp.float32),
                   jax.ShapeDtypeStruct((NP, 16), jnp.float32),
                   jax.ShapeDtypeStruct((NP, 16), jnp.float32),
                   jax.ShapeDtypeStruct((8, 16), jnp.float32)],
    )(xp, w, was, wad)


def _tc_project2(p0, p1, d0, d1, bexp, wa, wb, was, was2, wad, wad2):
    def body(p0_ref, p1_ref, d0_ref, d1_ref, bexp_ref, wa_ref, wb_ref,
             was_ref, was2_ref, wad_ref, wad2_ref,
             tha_ref, thb_ref, tas_ref, tad_ref, sh_ref):
        i = pl.program_id(0)
        inv = 1.0 / jnp.maximum(d0_ref[...] + d1_ref[...], 1e-30)
        scale = jnp.dot(inv, bexp_ref[...],
                        preferred_element_type=jnp.float32, precision=lax.Precision.HIGHEST)
        v = (p0_ref[...] + p1_ref[...]) * scale
        zb = jnp.where(v > 0, v, jnp.exp(v) - 1.0)
        tha = jnp.dot(zb, wa_ref[...], preferred_element_type=jnp.float32)
        thb = jnp.dot(zb, wb_ref[...], preferred_element_type=jnp.float32)
        tha_ref[...] = tha
        thb_ref[...] = thb
        tas = (jnp.dot(tha, was_ref[...], preferred_element_type=jnp.float32,
                       precision=lax.Precision.HIGHEST)
               + jnp.dot(thb, was2_ref[...],
                         preferred_element_type=jnp.float32,
                         precision=lax.Precision.HIGHEST))
        tad = (jnp.dot(tha, wad_ref[...], preferred_element_type=jnp.float32,
                       precision=lax.Precision.HIGHEST)
               + jnp.dot(thb, wad2_ref[...],
                         preferred_element_type=jnp.float32,
                         precision=lax.Precision.HIGHEST))
        tas_ref[...] = tas
        tad_ref[...] = tad
        _shift_update(i, tas, tad, sh_ref)

    return pl.pallas_call(
        body,
        grid=(_NG,),
        in_specs=[pl.BlockSpec((_GB, 96), lambda i: (i, 0)),
                  pl.BlockSpec((_GB, 96), lambda i: (i, 0)),
                  pl.BlockSpec((_GB, 16), lambda i: (i, 0)),
                  pl.BlockSpec((_GB, 16), lambda i: (i, 0)),
                  pl.BlockSpec((16, 96), lambda i: (0, 0)),
                  pl.BlockSpec((96, 96), lambda i: (0, 0)),
                  pl.BlockSpec((96, 96), lambda i: (0, 0)),
                  pl.BlockSpec((96, 16), lambda i: (0, 0)),
                  pl.BlockSpec((96, 16), lambda i: (0, 0)),
                  pl.BlockSpec((96, 16), lambda i: (0, 0)),
                  pl.BlockSpec((96, 16), lambda i: (0, 0))],
        out_specs=[pl.BlockSpec((_GB, 96), lambda i: (i, 0)),
                   pl.BlockSpec((_GB, 96), lambda i: (i, 0)),
                   pl.BlockSpec((_GB, 16), lambda i: (i, 0)),
                   pl.BlockSpec((_GB, 16), lambda i: (i, 0)),
                   pl.BlockSpec((8, 16), lambda i: (0, 0))],
        out_shape=[jax.ShapeDtypeStruct((NP, 96), jnp.float32),
                   jax.ShapeDtypeStruct((NP, 96), jnp.float32),
                   jax.ShapeDtypeStruct((NP, 16), jnp.float32),
                   jax.ShapeDtypeStruct((NP, 16), jnp.float32),
                   jax.ShapeDtypeStruct((8, 16), jnp.float32)],
    )(p0, p1, d0, d1, bexp, wa, wb, was, was2, wad, wad2)


def _tc_head(q0a, q1a, q0b, q1b, d0, d1, bexpa, bexpb, wdt, bd2):
    def body(q0a_ref, q1a_ref, q0b_ref, q1b_ref, d0_ref, d1_ref,
             bexpa_ref, bexpb_ref, wd_ref, bd_ref, res_ref, acc_ref):
        i = pl.program_id(0)

        @pl.when(i == 0)
        def _():
            acc_ref[...] = jnp.zeros((8, 192), jnp.float32)

        inv = 1.0 / jnp.maximum(d0_ref[...] + d1_ref[...], 1e-30)
        sca = jnp.dot(inv, bexpa_ref[...],
                      preferred_element_type=jnp.float32, precision=lax.Precision.HIGHEST)
        scb = jnp.dot(inv, bexpb_ref[...],
                      preferred_element_type=jnp.float32, precision=lax.Precision.HIGHEST)
        pa = jnp.sum((q0a_ref[...] + q1a_ref[...]) * sca, axis=0)
        pb = jnp.sum((q0b_ref[...] + q1b_ref[...]) * scb, axis=0)
        p = jnp.concatenate([pa, pb])[None, :]
        acc_ref[0:1, :] = acc_ref[0:1, :] + p

        @pl.when(i == _NG - 1)
        def _():
            g = acc_ref[0:1, :]
            nrm = jnp.sqrt(jnp.sum(g * g))
            gn = g / jnp.maximum(nrm, 1e-12)
            res_ref[...] = (jnp.sum(gn * wd_ref[...], axis=1, keepdims=True)
                            + bd_ref[...])

    return pl.pallas_call(
        body,
        grid=(_NG,),
        in_specs=[pl.BlockSpec((_GB, 96), lambda i: (i, 0)),
                  pl.BlockSpec((_GB, 96), lambda i: (i, 0)),
                  pl.BlockSpec((_GB, 96), lambda i: (i, 0)),
                  pl.BlockSpec((_GB, 96), lambda i: (i, 0)),
                  pl.BlockSpec((_GB, 16), lambda i: (i, 0)),
                  pl.BlockSpec((_GB, 16), lambda i: (i, 0)),
                  pl.BlockSpec((16, 96), lambda i: (0, 0)),
                  pl.BlockSpec((16, 96), lambda i: (0, 0)),
                  pl.BlockSpec((1, 192), lambda i: (0, 0)),
                  pl.BlockSpec((1, 1), lambda i: (0, 0))],
        out_specs=pl.BlockSpec((1, 1), lambda i: (0, 0)),
        out_shape=jax.ShapeDtypeStruct((1, 1), jnp.float32),
        scratch_shapes=[pltpu.VMEM((8, 192), jnp.float32)],
    )(q0a, q1a, q0b, q1b, d0, d1, bexpa, bexpb, wdt, bd2)


def _headmix(a):
    h, f = a.shape
    eye = jnp.eye(h, dtype=a.dtype)
    return (a[:, :, None] * eye[:, None, :]).reshape(h * f, h)


def kernel(x, edge_index, W1, a1_src, a1_dst, W2, a2_src, a2_dst, Wd, bd):
    f32 = jnp.float32
    src = edge_index[0].astype(jnp.int32)
    dst = edge_index[1].astype(jnp.int32)
    ppt = EPT - E // 32
    epad = jnp.broadcast_to(N + jnp.arange(ppt, dtype=jnp.int32), (32, ppt))
    srcp = jnp.concatenate([src.reshape(32, E // 32), epad], axis=1).reshape(-1)
    dstp = jnp.concatenate([dst.reshape(32, E // 32), epad], axis=1).reshape(-1)
    xp = jnp.pad(x, ((0, NP - N), (0, 5)))

    W1p = jnp.pad(W1, ((0, 5), (0, 0)))
    A1s = jnp.pad(_headmix(a1_src), ((0, 0), (0, 10)))
    A1d = jnp.pad(_headmix(a1_dst), ((0, 0), (0, 10)))

    z16 = jnp.zeros((SB, 16), f32)
    z96 = jnp.zeros((SB, 96), f32)
    eye6 = jnp.eye(6, dtype=f32)
    bexp96 = jnp.pad(jnp.kron(eye6, jnp.ones((1, 16), f32)), ((0, 10), (0, 0)))
    k32 = jnp.kron(eye6, jnp.ones((1, 32), f32))
    bexpa = jnp.pad(k32[:, :96], ((0, 10), (0, 0)))
    bexpb = jnp.pad(k32[:, 96:], ((0, 10), (0, 0)))

    th1, tas1, tad1, shift1 = _tc_project1(xp, W1p, A1s, A1d)
    den1, out1 = _sc_fused_l1(tas1, tad1, th1, srcp, dstp, shift1, z16, z96)

    A2s = _headmix(a2_src)
    A2d = _headmix(a2_dst)
    A2sa = jnp.pad(A2s[:96], ((0, 0), (0, 10)))
    A2sb = jnp.pad(A2s[96:], ((0, 0), (0, 10)))
    A2da = jnp.pad(A2d[:96], ((0, 0), (0, 10)))
    A2db = jnp.pad(A2d[96:], ((0, 0), (0, 10)))

    tha, thb, tas2, tad2, shift2 = _tc_project2(out1[0], out1[1],
                                                den1[0], den1[1], bexp96,
                                                W2[:, :96], W2[:, 96:],
                                                A2sa, A2sb, A2da, A2db)
    den2, out2a, eexp2 = _sc_fused_l2a(tas2, tad2, tha, srcp, dstp,
                                       shift2, z16, z96)
    out2b = _sc_pass2_l2b(thb, srcp, dstp, eexp2, z96)

    res = _tc_head(out2a[0], out2a[1], out2b[0], out2b[1],
                   den2[0], den2[1], bexpa, bexpb,
                   Wd.T, bd.reshape(1, 1))
    return res.reshape(1)


# speedup vs baseline: 1.0146x; 1.0146x over previous
import functools

import jax
import jax.numpy as jnp
from jax import lax
from jax.experimental import pallas as pl
from jax.experimental.pallas import tpu as pltpu
from jax.experimental.pallas import tpu_sc as plsc

N = 10000
E = 320000
NP = 10240
G1 = 160
SB = 128
EPT = 40 * 256
EPAD = EPT * 32
RPT = NP // 16

_mesh = plsc.VectorSubcoreMesh(core_axis_name="c", subcore_axis_name="s")
_sc_params = pltpu.CompilerParams(use_tc_tiling_on_sc=False,
                                  needs_layout_passes=False)


def _make_fused(h0, nh, HW, emit_e):
    F = nh * HW * 16
    G = G1
    outs = [jax.ShapeDtypeStruct((2, NP, 16), jnp.float32),
            jax.ShapeDtypeStruct((2, NP, F), jnp.float32)]
    if emit_e:
        outs.append(jax.ShapeDtypeStruct((EPAD, 16), jnp.float32))

    buf = (pltpu.VMEM((G,), jnp.int32),
           pltpu.VMEM((G,), jnp.int32),
           pltpu.VMEM((G, 16), jnp.float32),
           pltpu.VMEM((G, 16), jnp.float32),
           pltpu.VMEM((G, 16), jnp.float32),
           pltpu.VMEM((G, F), jnp.float32),
           pltpu.SemaphoreType.DMA,
           pltpu.SemaphoreType.DMA,
           pltpu.SemaphoreType.DMA)

    @functools.partial(
        pl.kernel,
        out_type=tuple(outs),
        mesh=_mesh,
        scratch_types=buf + buf + (
            pltpu.VMEM((16,), jnp.float32),
            pltpu.VMEM((SB, 16), jnp.float32),
            pltpu.VMEM_SHARED((NP, 16), jnp.float32),
            pltpu.VMEM_SHARED((NP, F), jnp.float32),
        ),
        compiler_params=_sc_params,
    )
    def fused(*refs):
        if emit_e:
            (tas, tad, th, src, dst, shift, zeros16, zerosF,
             dout, oout, eout, *scr) = refs
        else:
            (tas, tad, th, src, dst, shift, zeros16, zerosF,
             dout, oout, *scr) = refs
        bufA, bufB = scr[0:9], scr[9:18]
        shv, stg, den, acc = scr[18:22]
        c = lax.axis_index("c")
        s = lax.axis_index("s")
        wid = c * 16 + s
        r0 = s * RPT
        srows0 = bufA[5]
        pltpu.sync_copy(zeros16, stg)
        pltpu.sync_copy(zerosF, srows0.at[pl.ds(0, SB)])
        for k in range(RPT // SB):
            pltpu.sync_copy(stg, den.at[pl.ds(r0 + k * SB, SB)])
            pltpu.sync_copy(srows0.at[pl.ds(0, SB)],
                            acc.at[pl.ds(r0 + k * SB, SB)])
        pltpu.sync_copy(shift.at[0], shv)
        plsc.subcore_barrier()
        shvv = shv[...]
        ebase = wid * EPT

        def issue(b, buf):
            sidx, didx, asr, adr, eex, srows, s1, s2, s3 = buf
            pltpu.sync_copy(src.at[pl.ds(b, G)], sidx)
            pltpu.sync_copy(dst.at[pl.ds(b, G)], didx)
            c3 = pltpu.async_copy(th.at[sidx], srows, s3)
            c1 = pltpu.async_copy(tas.at[sidx], asr, s1)
            c2 = pltpu.async_copy(tad.at[didx], adr, s2)
            return c1, c2, c3

        def process(b, buf, cps):
            sidx, didx, asr, adr, eex, srows, s1, s2, s3 = buf
            c1, c2, c3 = cps
            c1.wait()
            c2.wait()

            @plsc.parallel_loop(0, G, 1, unroll=4)
            def _edge_e(g):
                pre = asr[g] + adr[g]
                lr = jnp.maximum(pre, 0.2 * pre)
                eex[g] = jnp.exp(lr - shvv)

            c3.wait()

            @plsc.parallel_loop(0, G, 1, unroll=4)
            def _edge_w(g):
                ev = eex[g]
                for h in range(nh):
                    ws = jnp.broadcast_to(ev[h0 + h], (16,))
                    for q in range(HW):
                        sl = pl.ds((h * HW + q) * 16, 16)
                        srows[g, sl] = srows[g, sl] * ws

            if emit_e:
                pltpu.sync_copy(eex, eout.at[pl.ds(b, G)])
            pltpu.sync_copy(eex, den.at[didx], add=True)
            pltpu.sync_copy(srows, acc.at[didx], add=True)

        def pair(i, _):
            b0 = ebase + 2 * i * G
            b1 = b0 + G
            cpa = issue(b0, bufA)
            cpb = issue(b1, bufB)
            process(b0, bufA, cpa)
            process(b1, bufB, cpb)
            return 0

        lax.fori_loop(0, EPT // (2 * G), pair, 0)
        plsc.subcore_barrier()
        for k in range(RPT // SB):
            pltpu.sync_copy(den.at[pl.ds(r0 + k * SB, SB)], stg)
            pltpu.sync_copy(stg, dout.at[c, pl.ds(r0 + k * SB, SB)])
            pltpu.sync_copy(acc.at[pl.ds(r0 + k * SB, SB)],
                            srows0.at[pl.ds(0, SB)])
            pltpu.sync_copy(srows0.at[pl.ds(0, SB)],
                            oout.at[c, pl.ds(r0 + k * SB, SB)])

    return fused


_sc_fused_l1 = _make_fused(0, 6, 1, emit_e=False)
_sc_fused_l2a = _make_fused(0, 3, 2, emit_e=True)


def _make_pass2(h0, nh, HW, G):
    F = nh * HW * 16

    buf = (pltpu.VMEM((G,), jnp.int32),
           pltpu.VMEM((G,), jnp.int32),
           pltpu.VMEM((G, F), jnp.float32),
           pltpu.VMEM((G, 16), jnp.float32),
           pltpu.SemaphoreType.DMA)

    @functools.partial(
        pl.kernel,
        out_type=jax.ShapeDtypeStruct((2, NP, F), jnp.float32),
        mesh=_mesh,
        scratch_types=buf + buf + (
            pltpu.VMEM_SHARED((NP, F), jnp.float32),
        ),
        compiler_params=_sc_params,
    )
    def pass2(th, src, dst, eexp, zerosF, out, *scr):
        bufA, bufB = scr[0:5], scr[5:10]
        acc = scr[10]
        c = lax.axis_index("c")
        s = lax.axis_index("s")
        wid = c * 16 + s
        r0 = s * RPT
        srows0 = bufA[2]
        pltpu.sync_copy(zerosF, srows0.at[pl.ds(0, SB)])
        for k in range(RPT // SB):
            pltpu.sync_copy(srows0.at[pl.ds(0, SB)],
                            acc.at[pl.ds(r0 + k * SB, SB)])
        plsc.subcore_barrier()
        ebase = wid * EPT

        def issue(b, buf):
            sidx, didx, srows, eexc, sem = buf
            pltpu.sync_copy(src.at[pl.ds(b, G)], sidx)
            pltpu.sync_copy(dst.at[pl.ds(b, G)], didx)
            pltpu.sync_copy(eexp.at[pl.ds(b, G)], eexc)
            return pltpu.async_copy(th.at[sidx], srows, sem)

        def process(buf, cp):
            sidx, didx, srows, eexc, sem = buf
            cp.wait()

            @plsc.parallel_loop(0, G, 1, unroll=4)
            def _edge(g):
                ev = eexc[g]
                for h in range(nh):
                    ws = jnp.broadcast_to(ev[h0 + h], (16,))
                    for q in range(HW):
                        sl = pl.ds((h * HW + q) * 16, 16)
                        srows[g, sl] = srows[g, sl] * ws
            pltpu.sync_copy(srows, acc.at[didx], add=True)

        def pair(i, _):
            b0 = ebase + 2 * i * G
            cpa = issue(b0, bufA)
            cpb = issue(b0 + G, bufB)
            process(bufA, cpa)
            process(bufB, cpb)
            return 0

        lax.fori_loop(0, EPT // (2 * G), pair, 0)
        plsc.subcore_barrier()
        for k in range(RPT // SB):
            pltpu.sync_copy(acc.at[pl.ds(r0 + k * SB, SB)],
                            srows0.at[pl.ds(0, SB)])
            pltpu.sync_copy(srows0.at[pl.ds(0, SB)],
                            out.at[c, pl.ds(r0 + k * SB, SB)])

    return pass2


_sc_pass2_l2b = _make_pass2(3, 3, 2, 256)


_GB = 2048
_NG = NP // _GB


def _shift_update(i, tas, tad, sh_ref):
    ms = jnp.max(tas, axis=0)[None, :]
    md = jnp.max(tad, axis=0)[None, :]
    blk = jnp.concatenate([jnp.broadcast_to(ms, (4, 16)),
                           jnp.broadcast_to(md, (4, 16))], axis=0)

    @pl.when(i == 0)
    def _():
        sh_ref[...] = blk

    @pl.when(i > 0)
    def _():
        sh_ref[...] = jnp.maximum(sh_ref[...], blk)

    @pl.when(i == _NG - 1)
    def _():
        a = sh_ref[...]
        tot = a[0:4, :] + a[4:8, :]
        sh = jnp.maximum(tot, 0.2 * tot)
        sh_ref[...] = jnp.concatenate([sh, sh], axis=0)


def _tc_project1(xp, w, was, wad):
    def body(x_ref, w_ref, was_ref, wad_ref,
             th_ref, tas_ref, tad_ref, sh_ref):
        i = pl.program_id(0)
        xb = x_ref[...]
        th = jnp.dot(xb, w_ref[...], preferred_element_type=jnp.float32)
        th_ref[...] = th
        tas = jnp.dot(th, was_ref[...], preferred_element_type=jnp.float32,
                      precision=lax.Precision.HIGHEST)
        tad = jnp.dot(th, wad_ref[...], preferred_element_type=jnp.float32,
                      precision=lax.Precision.HIGHEST)
        tas_ref[...] = tas
        tad_ref[...] = tad
        _shift_update(i, tas, tad, sh_ref)

    return pl.pallas_call(
        body,
        grid=(_NG,),
        in_specs=[pl.BlockSpec((_GB, 16), lambda i: (i, 0)),
                  pl.BlockSpec((16, 96), lambda i: (0, 0)),
                  pl.BlockSpec((96, 16), lambda i: (0, 0)),
                  pl.BlockSpec((96, 16), lambda i: (0, 0))],
        out_specs=[pl.BlockSpec((_GB, 96), lambda i: (i, 0)),
                   pl.BlockSpec((_GB, 16), lambda i: (i, 0)),
                   pl.BlockSpec((_GB, 16), lambda i: (i, 0)),
                   pl.BlockSpec((8, 16), lambda i: (0, 0))],
        out_shape=[jax.ShapeDtypeStruct((NP, 96), jnp.float32),
                   jax.ShapeDtypeStruct((NP, 16), jnp.float32),
                   jax.ShapeDtypeStruct((NP, 16), jnp.float32),
                   jax.ShapeDtypeStruct((8, 16), jnp.float32)],
    )(xp, w, was, wad)


def _tc_project2(p0, p1, d0, d1, bexp, wa, wb, was, was2, wad, wad2):
    def body(p0_ref, p1_ref, d0_ref, d1_ref, bexp_ref, wa_ref, wb_ref,
             was_ref, was2_ref, wad_ref, wad2_ref,
             tha_ref, thb_ref, tas_ref, tad_ref, sh_ref):
        i = pl.program_id(0)
        inv = 1.0 / jnp.maximum(d0_ref[...] + d1_ref[...], 1e-30)
        scale = jnp.dot(inv, bexp_ref[...],
                        preferred_element_type=jnp.float32, precision=lax.Precision.HIGHEST)
        v = (p0_ref[...] + p1_ref[...]) * scale
        zb = jnp.where(v > 0, v, jnp.exp(v) - 1.0)
        tha = jnp.dot(zb, wa_ref[...], preferred_element_type=jnp.float32)
        thb = jnp.dot(zb, wb_ref[...], preferred_element_type=jnp.float32)
        tha_ref[...] = tha
        thb_ref[...] = thb
        tas = (jnp.dot(tha, was_ref[...], preferred_element_type=jnp.float32,
                       precision=lax.Precision.HIGHEST)
               + jnp.dot(thb, was2_ref[...],
                         preferred_element_type=jnp.float32,
                         precision=lax.Precision.HIGHEST))
        tad = (jnp.dot(tha, wad_ref[...], preferred_element_type=jnp.float32,
                       precision=lax.Precision.HIGHEST)
               + jnp.dot(thb, wad2_ref[...],
                         preferred_element_type=jnp.float32,
                         precision=lax.Precision.HIGHEST))
        tas_ref[...] = tas
        tad_ref[...] = tad
        _shift_update(i, tas, tad, sh_ref)

    return pl.pallas_call(
        body,
        grid=(_NG,),
        in_specs=[pl.BlockSpec((_GB, 96), lambda i: (i, 0)),
                  pl.BlockSpec((_GB, 96), lambda i: (i, 0)),
                  pl.BlockSpec((_GB, 16), lambda i: (i, 0)),
                  pl.BlockSpec((_GB, 16), lambda i: (i, 0)),
                  pl.BlockSpec((16, 96), lambda i: (0, 0)),
                  pl.BlockSpec((96, 96), lambda i: (0, 0)),
                  pl.BlockSpec((96, 96), lambda i: (0, 0)),
                  pl.BlockSpec((96, 16), lambda i: (0, 0)),
                  pl.BlockSpec((96, 16), lambda i: (0, 0)),
                  pl.BlockSpec((96, 16), lambda i: (0, 0)),
                  pl.BlockSpec((96, 16), lambda i: (0, 0))],
        out_specs=[pl.BlockSpec((_GB, 96), lambda i: (i, 0)),
                   pl.BlockSpec((_GB, 96), lambda i: (i, 0)),
                   pl.BlockSpec((_GB, 16), lambda i: (i, 0)),
                   pl.BlockSpec((_GB, 16), lambda i: (i, 0)),
                   pl.BlockSpec((8, 16), lambda i: (0, 0))],
        out_shape=[jax.ShapeDtypeStruct((NP, 96), jnp.float32),
                   jax.ShapeDtypeStruct((NP, 96), jnp.float32),
                   jax.ShapeDtypeStruct((NP, 16), jnp.float32),
                   jax.ShapeDtypeStruct((NP, 16), jnp.float32),
                   jax.ShapeDtypeStruct((8, 16), jnp.float32)],
    )(p0, p1, d0, d1, bexp, wa, wb, was, was2, wad, wad2)


def _tc_head(q0a, q1a, q0b, q1b, d0, d1, bexpa, bexpb, wdt, bd2):
    def body(q0a_ref, q1a_ref, q0b_ref, q1b_ref, d0_ref, d1_ref,
             bexpa_ref, bexpb_ref, wd_ref, bd_ref, res_ref, acc_ref):
        i = pl.program_id(0)

        @pl.when(i == 0)
        def _():
            acc_ref[...] = jnp.zeros((8, 192), jnp.float32)

        inv = 1.0 / jnp.maximum(d0_ref[...] + d1_ref[...], 1e-30)
        sca = jnp.dot(inv, bexpa_ref[...],
                      preferred_element_type=jnp.float32, precision=lax.Precision.HIGHEST)
        scb = jnp.dot(inv, bexpb_ref[...],
                      preferred_element_type=jnp.float32, precision=lax.Precision.HIGHEST)
        pa = jnp.sum((q0a_ref[...] + q1a_ref[...]) * sca, axis=0)
        pb = jnp.sum((q0b_ref[...] + q1b_ref[...]) * scb, axis=0)
        p = jnp.concatenate([pa, pb])[None, :]
        acc_ref[0:1, :] = acc_ref[0:1, :] + p

        @pl.when(i == _NG - 1)
        def _():
            g = acc_ref[0:1, :]
            nrm = jnp.sqrt(jnp.sum(g * g))
            gn = g / jnp.maximum(nrm, 1e-12)
            res_ref[...] = (jnp.sum(gn * wd_ref[...], axis=1, keepdims=True)
                            + bd_ref[...])

    return pl.pallas_call(
        body,
        grid=(_NG,),
        in_specs=[pl.BlockSpec((_GB, 96), lambda i: (i, 0)),
                  pl.BlockSpec((_GB, 96), lambda i: (i, 0)),
                  pl.BlockSpec((_GB, 96), lambda i: (i, 0)),
                  pl.BlockSpec((_GB, 96), lambda i: (i, 0)),
                  pl.BlockSpec((_GB, 16), lambda i: (i, 0)),
                  pl.BlockSpec((_GB, 16), lambda i: (i, 0)),
                  pl.BlockSpec((16, 96), lambda i: (0, 0)),
                  pl.BlockSpec((16, 96), lambda i: (0, 0)),
                  pl.BlockSpec((1, 192), lambda i: (0, 0)),
                  pl.BlockSpec((1, 1), lambda i: (0, 0))],
        out_specs=pl.BlockSpec((1, 1), lambda i: (0, 0)),
        out_shape=jax.ShapeDtypeStruct((1, 1), jnp.float32),
        scratch_shapes=[pltpu.VMEM((8, 192), jnp.float32)],
    )(q0a, q1a, q0b, q1b, d0, d1, bexpa, bexpb, wdt, bd2)


def _headmix(a):
    h, f = a.shape
    eye = jnp.eye(h, dtype=a.dtype)
    return (a[:, :, None] * eye[:, None, :]).reshape(h * f, h)


def kernel(x, edge_index, W1, a1_src, a1_dst, W2, a2_src, a2_dst, Wd, bd):
    f32 = jnp.float32
    src = edge_index[0].astype(jnp.int32)
    dst = edge_index[1].astype(jnp.int32)
    ppt = EPT - E // 32
    epad = jnp.broadcast_to(N + jnp.arange(ppt, dtype=jnp.int32), (32, ppt))
    srcp = jnp.concatenate([src.reshape(32, E // 32), epad], axis=1).reshape(-1)
    dstp = jnp.concatenate([dst.reshape(32, E // 32), epad], axis=1).reshape(-1)
    xp = jnp.pad(x, ((0, NP - N), (0, 5)))

    W1p = jnp.pad(W1, ((0, 5), (0, 0)))
    A1s = jnp.pad(_headmix(a1_src), ((0, 0), (0, 10)))
    A1d = jnp.pad(_headmix(a1_dst), ((0, 0), (0, 10)))

    z16 = jnp.zeros((SB, 16), f32)
    z96 = jnp.zeros((SB, 96), f32)
    eye6 = jnp.eye(6, dtype=f32)
    bexp96 = jnp.pad(jnp.kron(eye6, jnp.ones((1, 16), f32)), ((0, 10), (0, 0)))
    k32 = jnp.kron(eye6, jnp.ones((1, 32), f32))
    bexpa = jnp.pad(k32[:, :96], ((0, 10), (0, 0)))
    bexpb = jnp.pad(k32[:, 96:], ((0, 10), (0, 0)))

    th1, tas1, tad1, shift1 = _tc_project1(xp, W1p, A1s, A1d)
    den1, out1 = _sc_fused_l1(tas1, tad1, th1, srcp, dstp, shift1, z16, z96)

    A2s = _headmix(a2_src)
    A2d = _headmix(a2_dst)
    A2sa = jnp.pad(A2s[:96], ((0, 0), (0, 10)))
    A2sb = jnp.pad(A2s[96:], ((0, 0), (0, 10)))
    A2da = jnp.pad(A2d[:96], ((0, 0), (0, 10)))
    A2db = jnp.pad(A2d[96:], ((0, 0), (0, 10)))

    tha, thb, tas2, tad2, shift2 = _tc_project2(out1[0], out1[1],
                                                den1[0], den1[1], bexp96,
                                                W2[:, :96], W2[:, 96:],
                                                A2sa, A2sb, A2da, A2db)
    den2, out2a, eexp2 = _sc_fused_l2a(tas2, tad2, tha, srcp, dstp,
                                       shift2, z16, z96)
    out2b = _sc_pass2_l2b(thb, srcp, dstp, eexp2, z96)

    res = _tc_head(out2a[0], out2a[1], out2b[0], out2b[1],
                   den2[0], den2[1], bexpa, bexpb,
                   Wd.T, bd.reshape(1, 1))
    return res.reshape(1)
